# baseline (speedup 1.0000x reference)
import functools

import jax
import jax.numpy as jnp
from jax import lax
from jax.experimental import pallas as pl
from jax.experimental.pallas import tpu as pltpu
from jax.experimental.pallas import tpu_sc as plsc

_B = 16384
_EMB = 32
_NUMC = 13
_VOCABS = (1000,) * 6 + (100,) * 20
_NF = len(_VOCABS)
_NIDX = _NF * _B
_TOTV = sum(_VOCABS)

_WINDOW = 1024
_TBLK = 2048


_STARTS = tuple(sum(_VOCABS[:i]) for i in range(_NF))


def _cats_gather(tables, idx):
    mesh = plsc.VectorSubcoreMesh(core_axis_name="c", subcore_axis_name="s")

    @functools.partial(
        pl.kernel,
        out_type=jax.ShapeDtypeStruct((_NIDX, _EMB), jnp.float32),
        mesh=mesh,
        compiler_params=pltpu.CompilerParams(use_tc_tiling_on_sc=False),
        scratch_types=[pltpu.VMEM_SHARED((_TOTV, _EMB), jnp.float32)],
    )
    def k(*refs):
        tbls = refs[:_NF]
        idx_hbm = refs[_NF]
        out_hbm = refs[_NF + 1]
        tbl_sp = refs[_NF + 2]
        sid = lax.axis_index("s")

        for i in range(_NF):

            @pl.when(sid == i % 16)
            def _(i=i):
                pltpu.sync_copy(
                    tbls[i], tbl_sp.at[pl.ds(_STARTS[i], _VOCABS[i])]
                )

        plsc.subcore_barrier()

        def body(i_vmem, o_vmem):
            pltpu.sync_copy(tbl_sp.at[i_vmem], o_vmem)

        pltpu.emit_pipeline(
            body,
            grid=(_NIDX // _WINDOW,),
            in_specs=[pl.BlockSpec((_WINDOW,), lambda i: (i,))],
            out_specs=[pl.BlockSpec((_WINDOW, _EMB), lambda i: (i, 0))],
            core_axis_name=("c", "s"),
            dimension_semantics=(pltpu.PARALLEL,),
        )(idx_hbm, out_hbm)

    return k(*tables, idx)


def _prep(nums, cats):
    def body(*refs):
        n_refs = refs[:_NUMC]
        c_refs = refs[_NUMC:_NUMC + _NF]
        xn_ref = refs[_NUMC + _NF]
        idx_ref = refs[_NUMC + _NF + 1]
        xn_ref[...] = jnp.stack([r[...] for r in n_refs], axis=0).T
        f = jax.lax.broadcasted_iota(jnp.int32, (1, _NF), 1)
        off = jnp.where(f < 6, f * 1000, 6000 + (f - 6) * 100)
        idx_ref[...] = jnp.stack([r[...] for r in c_refs], axis=0).T + off

    return pl.pallas_call(
        body,
        grid=(_B // _TBLK,),
        in_specs=[pl.BlockSpec((_TBLK,), lambda j: (j,))
                  for _ in range(_NUMC + _NF)],
        out_specs=[
            pl.BlockSpec((_TBLK, _NUMC), lambda j: (j, 0)),
            pl.BlockSpec((_TBLK, _NF), lambda j: (j, 0)),
        ],
        out_shape=[
            jax.ShapeDtypeStruct((_B, _NUMC), jnp.float32),
            jax.ShapeDtypeStruct((_B, _NF), jnp.int32),
        ],
    )(*nums, *cats)


def kernel(num_0, num_1, num_2, num_3, num_4, num_5, num_6, num_7, num_8,
           num_9, num_10, num_11, num_12,
           cat_0, cat_1, cat_2, cat_3, cat_4, cat_5, cat_6, cat_7, cat_8,
           cat_9, cat_10, cat_11, cat_12, cat_13, cat_14, cat_15, cat_16,
           cat_17, cat_18, cat_19, cat_20, cat_21, cat_22, cat_23, cat_24,
           cat_25,
           W_0, W_1, W_2, W_3, W_4, W_5, W_6, W_7, W_8, W_9, W_10, W_11,
           W_12, W_13, W_14, W_15, W_16, W_17, W_18, W_19, W_20, W_21,
           W_22, W_23, W_24, W_25):
    nums = [num_0, num_1, num_2, num_3, num_4, num_5, num_6, num_7, num_8,
            num_9, num_10, num_11, num_12]
    cats = [cat_0, cat_1, cat_2, cat_3, cat_4, cat_5, cat_6, cat_7, cat_8,
            cat_9, cat_10, cat_11, cat_12, cat_13, cat_14, cat_15, cat_16,
            cat_17, cat_18, cat_19, cat_20, cat_21, cat_22, cat_23, cat_24,
            cat_25]
    tables = [W_0, W_1, W_2, W_3, W_4, W_5, W_6, W_7, W_8, W_9, W_10, W_11,
              W_12, W_13, W_14, W_15, W_16, W_17, W_18, W_19, W_20, W_21,
              W_22, W_23, W_24, W_25]

    x_nums, idx = _prep(nums, cats)

    gathered = _cats_gather(tables, idx.reshape(_NIDX))
    x_cats = gathered.reshape(_B, _NF * _EMB)
    return (x_nums, x_cats)
